# baseline (speedup 1.0000x reference)
import jax
import jax.numpy as jnp
from jax.experimental import pallas as pl

B, N, F = 32, 512, 128
GCN_UNITS = 32
DENSE_UNITS = 512
GPB = 8


def _gcn_pool_kernel(x_ref, a_ref, wg_ref, bg_ref, out_ref):
    bg = bg_ref[0, :]
    for g in range(GPB):
        xw = jnp.dot(x_ref[g], wg_ref[:, :], preferred_element_type=jnp.float32)
        h = jnp.dot(a_ref[g], xw, preferred_element_type=jnp.float32)
        out_ref[g, 0, :] = jnp.mean(h, axis=0) + bg
        out_ref[g, 1, :] = jnp.max(h, axis=0) + bg


def _head_kernel(p_ref, w1_ref, b1_ref, w2_ref, b2_ref, out_ref):
    p = p_ref[:, :, :].reshape(B, 2 * GCN_UNITS)
    z = jnp.dot(p, w1_ref[:, :], preferred_element_type=jnp.float32)
    z = jnp.maximum(z + b1_ref[0, :], 0.0)
    out = jnp.dot(z, w2_ref[:, :], preferred_element_type=jnp.float32)
    out_ref[:, :] = out + b2_ref[0, :]


@jax.jit
def kernel(x, a, W_gcn, b_gcn, W1, b1, W2, b2):
    pooled = pl.pallas_call(
        _gcn_pool_kernel,
        grid=(B // GPB,),
        in_specs=[
            pl.BlockSpec((GPB, N, F), lambda b: (b, 0, 0)),
            pl.BlockSpec((GPB, N, N), lambda b: (b, 0, 0)),
            pl.BlockSpec((F, GCN_UNITS), lambda b: (0, 0)),
            pl.BlockSpec((1, GCN_UNITS), lambda b: (0, 0)),
        ],
        out_specs=pl.BlockSpec((GPB, 2, GCN_UNITS), lambda b: (b, 0, 0)),
        out_shape=jax.ShapeDtypeStruct((B, 2, GCN_UNITS), jnp.float32),
    )(x[..., :F], a, W_gcn, b_gcn.reshape(1, GCN_UNITS))

    out = pl.pallas_call(
        _head_kernel,
        grid=(1,),
        in_specs=[
            pl.BlockSpec((B, 2, GCN_UNITS), lambda i: (0, 0, 0)),
            pl.BlockSpec((2 * GCN_UNITS, DENSE_UNITS), lambda i: (0, 0)),
            pl.BlockSpec((1, DENSE_UNITS), lambda i: (0, 0)),
            pl.BlockSpec((DENSE_UNITS, 1), lambda i: (0, 0)),
            pl.BlockSpec((1, 1), lambda i: (0, 0)),
        ],
        out_specs=pl.BlockSpec((B, 1), lambda i: (0, 0)),
        out_shape=jax.ShapeDtypeStruct((B, 1), jnp.float32),
    )(pooled, W1, b1.reshape(1, DENSE_UNITS), W2, b2.reshape(1, 1))
    return out
